# baseline (speedup 1.0000x reference)
import functools

import jax
import jax.numpy as jnp
from jax import lax
from jax.experimental import pallas as pl
from jax.experimental.pallas import tpu as pltpu
from jax.experimental.pallas import tpu_sc as plsc

K = 16
MARGIN = 1.0
N = 10000
NPAD = 10240
D = 256
A = 2048
BA = 128
BISECT = 7


def _sc_gather_anchors(out1, out2, anchor1, anchor2):
    info = plsc.get_sparse_core_info()
    nw = info.num_cores * info.num_subcores
    bpw = A // nw
    mesh = plsc.VectorSubcoreMesh(core_axis_name="c", subcore_axis_name="s")

    @functools.partial(
        pl.kernel,
        mesh=mesh,
        out_type=[
            jax.ShapeDtypeStruct((A, D), jnp.float32),
            jax.ShapeDtypeStruct((A, D), jnp.float32),
        ],
        scratch_types=[
            pltpu.VMEM((bpw,), jnp.int32),
            pltpu.VMEM((bpw, D), jnp.float32),
            pltpu.SemaphoreType.DMA,
        ],
    )
    def gather_kernel(t1_hbm, t2_hbm, i1_hbm, i2_hbm, o1_hbm, o2_hbm,
                      idx_v, rows_v, sem):
        wid = lax.axis_index("s") * info.num_cores + lax.axis_index("c")
        base = wid * bpw
        pltpu.sync_copy(i1_hbm.at[pl.ds(base, bpw)], idx_v)
        pltpu.async_copy(t1_hbm.at[idx_v], rows_v, sem).wait()
        pltpu.sync_copy(rows_v, o1_hbm.at[pl.ds(base, bpw)])
        pltpu.sync_copy(i2_hbm.at[pl.ds(base, bpw)], idx_v)
        pltpu.async_copy(t2_hbm.at[idx_v], rows_v, sem).wait()
        pltpu.sync_copy(rows_v, o2_hbm.at[pl.ds(base, bpw)])

    return gather_kernel(out1, out2, anchor1.astype(jnp.int32),
                         anchor2.astype(jnp.int32))


def _tc_loss_body(out2p_ref, out1p_ref, ae1_ref, ae2_ref, out_ref,
                  simt0_ref, simt1_ref, rinv2_ref, rinv1_ref):
    a_blk = pl.program_id(0)

    @pl.when(a_blk == 0)
    def _norms():
        for tref, rref in ((out2p_ref, rinv2_ref), (out1p_ref, rinv1_ref)):
            t = tref[...]
            rref[...] = lax.rsqrt(jnp.sum(t * t, axis=1, keepdims=True))

    ae1 = ae1_ref[...]
    ae2 = ae2_ref[...]
    n1sq = jnp.sum(ae1 * ae1, axis=1, keepdims=True)
    n2sq = jnp.sum(ae2 * ae2, axis=1, keepdims=True)
    ae1n = ae1 * lax.rsqrt(n1sq)
    ae2n = ae2 * lax.rsqrt(n2sq)

    c = 1.0 - jnp.sum(ae1n * ae2n, axis=1)

    row_ids = lax.broadcasted_iota(jnp.int32, (NPAD, 1), 0)
    neg = jnp.float32(-1e30)
    ones8 = jnp.ones((8, NPAD), jnp.bfloat16)
    one_b = jnp.bfloat16(1.0)
    zero_b = jnp.bfloat16(0.0)

    def colsum(mat_b):
        r8 = lax.dot_general(ones8, mat_b, (((1,), (0,)), ((), ())),
                             preferred_element_type=jnp.float32)
        return jnp.max(r8, axis=0)

    srefs = (simt0_ref, simt1_ref)
    for side in range(2):
        tbl = out2p_ref[...] if side == 0 else out1p_ref[...]
        rinv = rinv2_ref[...] if side == 0 else rinv1_ref[...]
        anc = ae1n if side == 0 else ae2n
        simt = lax.dot_general(
            tbl, anc, (((1,), (1,)), ((), ())),
            preferred_element_type=jnp.float32)
        simt = simt * rinv
        srefs[side][...] = jnp.where(row_ids < N, simt, neg
                                     ).astype(jnp.bfloat16)

    lo = [jnp.full((BA,), -1.05, jnp.float32) for _ in range(2)]
    hi = [jnp.full((BA,), 1.05, jnp.float32) for _ in range(2)]
    for _ in range(BISECT):
        t = [0.5 * (lo[i] + hi[i]) for i in range(2)]
        cnt = [colsum(jnp.where(
                   srefs[i][...] > t[i].astype(jnp.bfloat16)[None, :],
                   one_b, zero_b))
               for i in range(2)]
        for i in range(2):
            ge16 = cnt[i] >= K
            lo[i] = jnp.where(ge16, t[i], lo[i])
            hi[i] = jnp.where(ge16, hi[i], t[i])

    total = jnp.float32(0.0)
    for i in range(2):
        big = jnp.maximum(lo[i], -c).astype(jnp.bfloat16)
        s = srefs[i][...]
        msk = s > big[None, :]
        sv = jnp.where(msk, s, zero_b)
        iv = jnp.where(msk, one_b, zero_b)
        ssum = colsum(sv)
        cnt = colsum(iv)
        corr = jnp.maximum(cnt - K, 0.0) * jnp.maximum(c + lo[i], 0.0)
        total = total + jnp.sum(c * cnt + ssum - corr)

    @pl.when(a_blk == 0)
    def _init():
        out_ref[0, 0] = 0.0

    out_ref[0, 0] += total / (A * K)


def _tc_loss(out2p, out1p, ae1, ae2):
    grid = (A // BA,)
    return pl.pallas_call(
        _tc_loss_body,
        grid=grid,
        in_specs=[
            pl.BlockSpec((NPAD, D), lambda a: (0, 0)),
            pl.BlockSpec((NPAD, D), lambda a: (0, 0)),
            pl.BlockSpec((BA, D), lambda a: (a, 0)),
            pl.BlockSpec((BA, D), lambda a: (a, 0)),
        ],
        out_specs=pl.BlockSpec(memory_space=pltpu.SMEM),
        out_shape=jax.ShapeDtypeStruct((1, 1), jnp.float32),
        scratch_shapes=[
            pltpu.VMEM((NPAD, BA), jnp.bfloat16),
            pltpu.VMEM((NPAD, BA), jnp.bfloat16),
            pltpu.VMEM((NPAD, 1), jnp.float32),
            pltpu.VMEM((NPAD, 1), jnp.float32),
        ],
    )(out2p, out1p, ae1, ae2)


def kernel(out1, out2, anchor1, anchor2):
    ae1, ae2 = _sc_gather_anchors(out1, out2, anchor1, anchor2)
    pad = ((0, NPAD - N), (0, 0))
    out1p = jnp.pad(out1, pad)
    out2p = jnp.pad(out2, pad)
    loss = _tc_loss(out2p, out1p, ae1, ae2)
    return loss[0, 0]


# speedup vs baseline: 1.1982x; 1.1982x over previous
import functools

import jax
import jax.numpy as jnp
from jax import lax
from jax.experimental import pallas as pl
from jax.experimental.pallas import tpu as pltpu
from jax.experimental.pallas import tpu_sc as plsc

K = 16
MARGIN = 1.0
N = 10000
NPAD = 10240
D = 256
A = 2048
BA = 128
BISECT = 7


def _sc_gather_anchors(out1, out2, anchor1, anchor2):
    info = plsc.get_sparse_core_info()
    nw = info.num_cores * info.num_subcores
    bpw = A // nw
    mesh = plsc.VectorSubcoreMesh(core_axis_name="c", subcore_axis_name="s")

    @functools.partial(
        pl.kernel,
        mesh=mesh,
        out_type=[
            jax.ShapeDtypeStruct((A, D), jnp.float32),
            jax.ShapeDtypeStruct((A, D), jnp.float32),
        ],
        scratch_types=[
            pltpu.VMEM((bpw,), jnp.int32),
            pltpu.VMEM((bpw, D), jnp.float32),
            pltpu.SemaphoreType.DMA,
        ],
    )
    def gather_kernel(t1_hbm, t2_hbm, i1_hbm, i2_hbm, o1_hbm, o2_hbm,
                      idx_v, rows_v, sem):
        wid = lax.axis_index("s") * info.num_cores + lax.axis_index("c")
        base = wid * bpw
        pltpu.sync_copy(i1_hbm.at[pl.ds(base, bpw)], idx_v)
        pltpu.async_copy(t1_hbm.at[idx_v], rows_v, sem).wait()
        pltpu.sync_copy(rows_v, o1_hbm.at[pl.ds(base, bpw)])
        pltpu.sync_copy(i2_hbm.at[pl.ds(base, bpw)], idx_v)
        pltpu.async_copy(t2_hbm.at[idx_v], rows_v, sem).wait()
        pltpu.sync_copy(rows_v, o2_hbm.at[pl.ds(base, bpw)])

    return gather_kernel(out1, out2, anchor1.astype(jnp.int32),
                         anchor2.astype(jnp.int32))


def _tc_loss_body(out2p_ref, out1p_ref, ae1_ref, ae2_ref, out_ref,
                  simt0_ref, simt1_ref, rinv2_ref, rinv1_ref):
    a_blk = pl.program_id(0)

    @pl.when(a_blk == 0)
    def _norms():
        for tref, rref in ((out2p_ref, rinv2_ref), (out1p_ref, rinv1_ref)):
            t = tref[...]
            rref[...] = lax.rsqrt(jnp.sum(t * t, axis=1, keepdims=True))

    ae1 = ae1_ref[...]
    ae2 = ae2_ref[...]
    n1sq = jnp.sum(ae1 * ae1, axis=1, keepdims=True)
    n2sq = jnp.sum(ae2 * ae2, axis=1, keepdims=True)
    ae1n = ae1 * lax.rsqrt(n1sq)
    ae2n = ae2 * lax.rsqrt(n2sq)

    c = 1.0 - jnp.sum(ae1n * ae2n, axis=1)

    ones8 = jnp.ones((8, N), jnp.float32)

    def colsum(mat):
        r8 = lax.dot_general(ones8, mat, (((1,), (0,)), ((), ())),
                             preferred_element_type=jnp.float32)
        return jnp.max(r8, axis=0)

    srefs = (simt0_ref, simt1_ref)
    for side in range(2):
        tbl = out2p_ref[...] if side == 0 else out1p_ref[...]
        rinv = rinv2_ref[...] if side == 0 else rinv1_ref[...]
        anc = ae1n if side == 0 else ae2n
        simt = lax.dot_general(
            tbl, anc, (((1,), (1,)), ((), ())),
            preferred_element_type=jnp.float32)
        srefs[side][...] = simt * rinv

    lo = [jnp.full((BA,), -1.05, jnp.float32) for _ in range(2)]
    hi = [jnp.full((BA,), 1.05, jnp.float32) for _ in range(2)]
    for _ in range(BISECT):
        t = [0.5 * (lo[i] + hi[i]) for i in range(2)]
        cnt = [colsum(jnp.where(srefs[i][...] > t[i][None, :], 1.0, 0.0))
               for i in range(2)]
        for i in range(2):
            ge16 = cnt[i] >= K
            lo[i] = jnp.where(ge16, t[i], lo[i])
            hi[i] = jnp.where(ge16, hi[i], t[i])

    total = jnp.float32(0.0)
    for i in range(2):
        big = jnp.maximum(lo[i], -c)
        s = srefs[i][...]
        msk = s > big[None, :]
        sv = jnp.where(msk, s, 0.0)
        iv = jnp.where(msk, 1.0, 0.0)
        ssum = colsum(sv)
        cnt = colsum(iv)
        corr = jnp.maximum(cnt - K, 0.0) * jnp.maximum(c + lo[i], 0.0)
        total = total + jnp.sum(c * cnt + ssum - corr)

    @pl.when(a_blk == 0)
    def _init():
        out_ref[0, 0] = 0.0

    out_ref[0, 0] += total / (A * K)


def _tc_loss(out2p, out1p, ae1, ae2):
    grid = (A // BA,)
    return pl.pallas_call(
        _tc_loss_body,
        grid=grid,
        in_specs=[
            pl.BlockSpec((N, D), lambda a: (0, 0)),
            pl.BlockSpec((N, D), lambda a: (0, 0)),
            pl.BlockSpec((BA, D), lambda a: (a, 0)),
            pl.BlockSpec((BA, D), lambda a: (a, 0)),
        ],
        out_specs=pl.BlockSpec(memory_space=pltpu.SMEM),
        out_shape=jax.ShapeDtypeStruct((1, 1), jnp.float32),
        scratch_shapes=[
            pltpu.VMEM((N, BA), jnp.float32),
            pltpu.VMEM((N, BA), jnp.float32),
            pltpu.VMEM((N, 1), jnp.float32),
            pltpu.VMEM((N, 1), jnp.float32),
        ],
    )(out2p, out1p, ae1, ae2)


def kernel(out1, out2, anchor1, anchor2):
    ae1, ae2 = _sc_gather_anchors(out1, out2, anchor1, anchor2)
    loss = _tc_loss(out2, out1, ae1, ae2)
    return loss[0, 0]
